# baseline (speedup 1.0000x reference)
import jax
import jax.numpy as jnp
from jax.experimental import pallas as pl
from jax.experimental.pallas import tpu as pltpu


def _body(x_ref, m_ref, o_ref):
    m_i = m_ref[...]
    xm = jnp.where(m_i != 0, x_ref[...], 0.0)
    s = jnp.sum(xm, axis=3, keepdims=True)
    cnt_i = jnp.sum(m_i, axis=3, keepdims=True)
    cnt = cnt_i.astype(jnp.float32)
    mean = jnp.where(cnt_i > 0, s / jnp.maximum(cnt, 1.0), 0.0)
    o_ref[...] = jnp.broadcast_to(mean, o_ref.shape)


def kernel(x, mask):
    B, L, K, C = x.shape
    xt = jnp.transpose(x, (0, 2, 3, 1))
    mt = jnp.transpose(mask, (0, 2, 3, 1))

    KB = 8
    grid = (B, K // KB)
    spec = pl.BlockSpec((1, KB, C, L), lambda b, k: (b, k, 0, 0))

    out = pl.pallas_call(
        _body,
        out_shape=jax.ShapeDtypeStruct((B, K, C, L), x.dtype),
        grid=grid,
        in_specs=[spec, spec],
        out_specs=spec,
        compiler_params=pltpu.CompilerParams(
            dimension_semantics=("parallel", "arbitrary"),
            vmem_limit_bytes=61 * 1024 * 1024,
        ),
        name="masked_mean_bcast",
    )(xt, mt)
    return jnp.transpose(out, (0, 3, 1, 2))
